# baseline (speedup 1.0000x reference)
import functools

import jax
import jax.numpy as jnp
from jax import lax
from jax.experimental import pallas as pl
from jax.experimental.pallas import tpu as pltpu
from jax.experimental.pallas import tpu_sc as plsc

NC = 2
NS = 16
NW = NC * NS

LANES = 16


@functools.lru_cache(maxsize=None)
def _build(BL: int, V: int, SEG: int, D: int):
    assert D == 2 * LANES
    rows_pw = BL // NW
    assert rows_pw * NW == BL
    seg_per_chunk = 8
    chunk = seg_per_chunk * SEG
    assert rows_pw % chunk == 0
    n_chunks = rows_pw // chunk
    n_pairs = n_chunks // 2
    assert n_pairs * 2 == n_chunks and n_pairs >= 2

    mesh = plsc.VectorSubcoreMesh(core_axis_name="c", subcore_axis_name="s")

    @functools.partial(
        pl.kernel,
        out_type=jax.ShapeDtypeStruct((BL, D), jnp.float32),
        mesh=mesh,
        compiler_params=pltpu.CompilerParams(
            use_tc_tiling_on_sc=False, needs_layout_passes=False),
        scratch_types=[
            pltpu.VMEM((chunk,), jnp.int32),
            pltpu.VMEM((chunk,), jnp.int32),
            pltpu.VMEM((chunk, D // 2), jnp.int32),
            pltpu.VMEM((chunk, D // 2), jnp.int32),
            pltpu.VMEM((chunk, D), jnp.float32),
            pltpu.VMEM((SEG, D), jnp.float32),
            pltpu.SemaphoreType.DMA,
            pltpu.SemaphoreType.DMA,
            pltpu.SemaphoreType.DMA,
        ],
    )
    def k(ids_hbm, tok_hbm, pos_hbm, out_hbm,
          idx0, idx1, rows0, rows1, res_v, pos_v, gsem0, gsem1, wsem):
        wid = lax.axis_index("s") * NC + lax.axis_index("c")
        base = wid * rows_pw
        pltpu.sync_copy(pos_hbm, pos_v)

        def fire_chunk(g, idx_v, rows_v, gsem):
            start = base + g * chunk
            pltpu.sync_copy(ids_hbm.at[pl.ds(start, chunk)], idx_v)
            pltpu.make_async_copy(tok_hbm.at[idx_v], rows_v, gsem).start()

        def wait_gathers(rows_v, gsem):
            pltpu.make_async_copy(
                tok_hbm.at[pl.ds(0, chunk)], rows_v, gsem).wait()

        def unpack_add(rows_v):
            def body(r, c):
                p0 = pos_v[r, 0:LANES]
                p1 = pos_v[r, LANES:D]
                for s in range(seg_per_chunk):
                    row = s * SEG + r
                    packed = plsc.bitcast(rows_v[row, 0:LANES], jnp.bfloat16)
                    h0, h1 = plsc.unpack(
                        packed, format=plsc.PackFormat.INTERLEAVED,
                        preferred_element_type=jnp.float32)
                    res_v[row, 0:LANES] = h0 + p0
                    res_v[row, LANES:D] = h1 + p1
                return c
            lax.fori_loop(0, SEG, body, 0)

        def wb_start(g):
            start = base + g * chunk
            pltpu.make_async_copy(
                res_v, out_hbm.at[pl.ds(start, chunk)], wsem).start()

        def wb_wait():
            pltpu.make_async_copy(
                res_v, out_hbm.at[pl.ds(base, chunk)], wsem).wait()

        fire_chunk(0, idx0, rows0, gsem0)

        def pair(g2, c):
            ge = 2 * g2

            fire_chunk(ge + 1, idx1, rows1, gsem1)
            wait_gathers(rows0, gsem0)

            @pl.when(g2 > 0)
            def _():
                wb_wait()
            unpack_add(rows0)
            wb_start(ge)

            @pl.when(g2 < n_pairs - 1)
            def _():
                fire_chunk(ge + 2, idx0, rows0, gsem0)
            wait_gathers(rows1, gsem1)
            wb_wait()
            unpack_add(rows1)
            wb_start(ge + 1)
            return c

        lax.fori_loop(0, n_pairs, pair, 0)
        wb_wait()

    return k


def kernel(input_ids, token_emb, pos_emb):
    Bv, Lv = input_ids.shape
    V, D = token_emb.shape
    BL = Bv * Lv
    ids_flat = input_ids.reshape(BL).astype(jnp.int32)
    pos = pos_emb[:Lv]
    tok_bf = token_emb.astype(jnp.bfloat16)
    tok_pairs = tok_bf.reshape(V, 2, D // 2).transpose(0, 2, 1)
    tok_packed = jax.lax.bitcast_convert_type(tok_pairs, jnp.int32)
    out = _build(BL, V, Lv, D)(ids_flat, tok_packed, pos)
    return out.reshape(Bv, Lv, D)


# speedup vs baseline: 1.2185x; 1.2185x over previous
import functools

import jax
import jax.numpy as jnp
from jax import lax
from jax.experimental import pallas as pl
from jax.experimental.pallas import tpu as pltpu
from jax.experimental.pallas import tpu_sc as plsc

NC = 2
NS = 16
NW = NC * NS

LANES = 16


@functools.lru_cache(maxsize=None)
def _build(BL: int, V: int, SEG: int, D: int):
    assert D == 2 * LANES
    rows_pw = BL // NW
    assert rows_pw * NW == BL
    seg_per_chunk = 8
    chunk = seg_per_chunk * SEG
    assert rows_pw % chunk == 0
    n_chunks = rows_pw // chunk
    n_pairs = n_chunks // 2
    assert n_pairs * 2 == n_chunks and n_pairs >= 2

    mesh = plsc.VectorSubcoreMesh(core_axis_name="c", subcore_axis_name="s")

    @functools.partial(
        pl.kernel,
        out_type=jax.ShapeDtypeStruct((BL, D), jnp.float32),
        mesh=mesh,
        compiler_params=pltpu.CompilerParams(
            use_tc_tiling_on_sc=False, needs_layout_passes=False),
        scratch_types=[
            pltpu.VMEM((chunk,), jnp.int32),
            pltpu.VMEM((chunk,), jnp.int32),
            pltpu.VMEM((chunk, D), jnp.float32),
            pltpu.VMEM((chunk, D), jnp.float32),
            pltpu.VMEM((SEG, D), jnp.float32),
            pltpu.SemaphoreType.DMA,
            pltpu.SemaphoreType.DMA,
            pltpu.SemaphoreType.DMA,
            pltpu.SemaphoreType.DMA,
        ],
    )
    def k(ids_hbm, tok_hbm, pos_hbm, out_hbm,
          idx0, idx1, rows0, rows1, pos_v, gsem0, gsem1, wsem0, wsem1):
        wid = lax.axis_index("s") * NC + lax.axis_index("c")
        base = wid * rows_pw
        pltpu.sync_copy(pos_hbm, pos_v)

        def fire_chunk(g, idx_v, rows_v, gsem):
            start = base + g * chunk
            pltpu.sync_copy(ids_hbm.at[pl.ds(start, chunk)], idx_v)
            pltpu.make_async_copy(tok_hbm.at[idx_v], rows_v, gsem).start()

        def wait_gathers(rows_v, gsem):
            pltpu.make_async_copy(
                tok_hbm.at[pl.ds(0, chunk)], rows_v, gsem).wait()

        def add_pos(rows_v):
            def body(r, c):
                p0 = pos_v[r, 0:LANES]
                p1 = pos_v[r, LANES:D]
                for s in range(seg_per_chunk):
                    row = s * SEG + r
                    rows_v[row, 0:LANES] = rows_v[row, 0:LANES] + p0
                    rows_v[row, LANES:D] = rows_v[row, LANES:D] + p1
                return c
            lax.fori_loop(0, SEG, body, 0)

        def wb_start(g, rows_v, wsem):
            start = base + g * chunk
            pltpu.make_async_copy(
                rows_v, out_hbm.at[pl.ds(start, chunk)], wsem).start()

        def wb_wait(rows_v, wsem):
            pltpu.make_async_copy(
                rows_v, out_hbm.at[pl.ds(base, chunk)], wsem).wait()

        fire_chunk(0, idx0, rows0, gsem0)

        def pair(g2, c):
            ge = 2 * g2

            @pl.when(g2 > 0)
            def _():
                wb_wait(rows1, wsem1)
            fire_chunk(ge + 1, idx1, rows1, gsem1)

            wait_gathers(rows0, gsem0)
            add_pos(rows0)
            wb_start(ge, rows0, wsem0)

            @pl.when(g2 < n_pairs - 1)
            def _():
                wb_wait(rows0, wsem0)
                fire_chunk(ge + 2, idx0, rows0, gsem0)

            wait_gathers(rows1, gsem1)
            add_pos(rows1)
            wb_start(ge + 1, rows1, wsem1)
            return c

        lax.fori_loop(0, n_pairs, pair, 0)
        wb_wait(rows0, wsem0)
        wb_wait(rows1, wsem1)

    return k


def kernel(input_ids, token_emb, pos_emb):
    Bv, Lv = input_ids.shape
    V, D = token_emb.shape
    BL = Bv * Lv
    ids_flat = input_ids.reshape(BL).astype(jnp.int32)
    pos = pos_emb[:Lv]
    out = _build(BL, V, Lv, D)(ids_flat, token_emb, pos)
    return out.reshape(Bv, Lv, D)
